# baseline (device time: 29273 ns/iter reference)
import jax
import jax.numpy as jnp
from jax import lax
from jax.experimental import pallas as pl
from jax.experimental.pallas import tpu as pltpu

N_DEV = 16
N_TOK = 512
D_IN = 256
D_OUT = 512
N_EXP = 32
EXP_PER_DEV = 2
ROWS = N_TOK // N_DEV


def kernel(x, router_W, route_idx, expert_W, shared_W):
    def body(x_ref, router_ref, idx_ref, ew_ref, sw_ref, out_ref,
             ps_ref, rs_buf, ag_ref,
             rs_send, rs_recv, ag_send, ag_recv):
        my = lax.axis_index("i")

        xf = x_ref[:]
        xb = xf.astype(jnp.bfloat16)
        scores = jnp.dot(xf, router_ref[:], preferred_element_type=jnp.float32)
        s_max = jnp.max(scores, axis=-1, keepdims=True)
        p = jnp.exp(scores - s_max)
        probs = p / jnp.sum(p, axis=-1, keepdims=True)

        eids = lax.broadcasted_iota(jnp.int32, (N_TOK, N_EXP), 1)
        partial = jnp.zeros((N_TOK, D_OUT), jnp.float32)
        for j in range(EXP_PER_DEV):
            e = my * EXP_PER_DEV + j
            w = jnp.sum(jnp.where(eids == e, probs, 0.0), axis=1, keepdims=True)
            gate = idx_ref[:] == e
            w = jnp.where(gate, w, 0.0)
            xw = xb * w.astype(jnp.bfloat16)
            partial = partial + jnp.dot(
                xw, ew_ref[j].astype(jnp.bfloat16),
                preferred_element_type=jnp.float32)
        ps_ref[:] = partial.astype(jnp.bfloat16)

        barrier_sem = pltpu.get_barrier_semaphore()
        for k in range(1, N_DEV):
            tgt = lax.rem(my + k, N_DEV)
            pl.semaphore_signal(barrier_sem, inc=1, device_id=(tgt,),
                                device_id_type=pl.DeviceIdType.MESH)
        pl.semaphore_wait(barrier_sem, N_DEV - 1)

        far_first = sorted(range(1, N_DEV),
                           key=lambda k: -min(k, N_DEV - k))
        near_first = sorted(range(1, N_DEV),
                            key=lambda k: min(k, N_DEV - k))

        for k in far_first:
            c = lax.rem(my + k, N_DEV)
            rdma = pltpu.make_async_remote_copy(
                src_ref=ps_ref.at[pl.ds(c * ROWS, ROWS), :],
                dst_ref=rs_buf.at[k - 1],
                send_sem=rs_send.at[k - 1],
                recv_sem=rs_recv.at[k - 1],
                device_id=(c,),
                device_id_type=pl.DeviceIdType.MESH,
            )
            rdma.start()

        out_ref[:] = jnp.dot(xb, sw_ref[:].astype(jnp.bfloat16),
                             preferred_element_type=jnp.float32)

        red = ps_ref[pl.ds(my * ROWS, ROWS), :].astype(jnp.float32)
        for k in near_first:
            wait = pltpu.make_async_remote_copy(
                src_ref=ps_ref.at[pl.ds(0, ROWS), :],
                dst_ref=rs_buf.at[k - 1],
                send_sem=rs_send.at[k - 1],
                recv_sem=rs_recv.at[k - 1],
                device_id=(my,),
                device_id_type=pl.DeviceIdType.MESH,
            )
            wait.wait_recv()
            red = red + rs_buf[k - 1].astype(jnp.float32)
        ag_ref[pl.ds(my * ROWS, ROWS), :] = red.astype(jnp.bfloat16)

        for k in far_first:
            tgt = lax.rem(my + k, N_DEV)
            rdma = pltpu.make_async_remote_copy(
                src_ref=ag_ref.at[pl.ds(my * ROWS, ROWS), :],
                dst_ref=ag_ref.at[pl.ds(my * ROWS, ROWS), :],
                send_sem=ag_send.at[k - 1],
                recv_sem=ag_recv.at[k - 1],
                device_id=(tgt,),
                device_id_type=pl.DeviceIdType.MESH,
            )
            rdma.start()

        my_rows = pl.ds(my * ROWS, ROWS)
        out_ref[my_rows, :] = out_ref[my_rows, :] + red

        for k in near_first:
            wait = pltpu.make_async_remote_copy(
                src_ref=ag_ref.at[pl.ds(0, ROWS), :],
                dst_ref=ag_ref.at[pl.ds(0, ROWS), :],
                send_sem=ag_send.at[k - 1],
                recv_sem=ag_recv.at[k - 1],
                device_id=(my,),
                device_id_type=pl.DeviceIdType.MESH,
            )
            wait.wait_recv()
            src = lax.rem(my - k + N_DEV, N_DEV)
            rows = pl.ds(src * ROWS, ROWS)
            out_ref[rows, :] = out_ref[rows, :] + ag_ref[rows, :].astype(
                jnp.float32)
        for k in range(1, N_DEV):
            wait = pltpu.make_async_remote_copy(
                src_ref=ag_ref.at[pl.ds(0, ROWS), :],
                dst_ref=ag_ref.at[pl.ds(0, ROWS), :],
                send_sem=ag_send.at[k - 1],
                recv_sem=ag_recv.at[k - 1],
                device_id=(my,),
                device_id_type=pl.DeviceIdType.MESH,
            )
            wait.wait_send()
            wait_a = pltpu.make_async_remote_copy(
                src_ref=ps_ref.at[pl.ds(0, ROWS), :],
                dst_ref=rs_buf.at[k - 1],
                send_sem=rs_send.at[k - 1],
                recv_sem=rs_recv.at[k - 1],
                device_id=(my,),
                device_id_type=pl.DeviceIdType.MESH,
            )
            wait_a.wait_send()

    return pl.pallas_call(
        body,
        out_shape=jax.ShapeDtypeStruct((N_TOK, D_OUT), jnp.float32),
        in_specs=[pl.BlockSpec(memory_space=pltpu.VMEM)] * 5,
        out_specs=pl.BlockSpec(memory_space=pltpu.VMEM),
        scratch_shapes=[
            pltpu.VMEM((N_TOK, D_OUT), jnp.bfloat16),
            pltpu.VMEM((N_DEV - 1, ROWS, D_OUT), jnp.bfloat16),
            pltpu.VMEM((N_TOK, D_OUT), jnp.bfloat16),
            pltpu.SemaphoreType.DMA((N_DEV - 1,)),
            pltpu.SemaphoreType.DMA((N_DEV - 1,)),
            pltpu.SemaphoreType.DMA((N_DEV - 1,)),
            pltpu.SemaphoreType.DMA((N_DEV - 1,)),
        ],
        compiler_params=pltpu.CompilerParams(collective_id=0),
    )(x, router_W, route_idx, expert_W, shared_W)


# device time: 24853 ns/iter; 1.1778x vs baseline; 1.1778x over previous
import jax
import jax.numpy as jnp
from jax import lax
from jax.experimental import pallas as pl
from jax.experimental.pallas import tpu as pltpu

N_DEV = 16
N_TOK = 512
D_IN = 256
D_OUT = 512
N_EXP = 32
EXP_PER_DEV = 2
ROWS = N_TOK // N_DEV


def kernel(x, router_W, route_idx, expert_W, shared_W):
    def body(x_ref, router_ref, idx_ref, ew_ref, sw_ref, out_ref,
             ps_ref, rs_buf, ag_ref,
             rs_send, rs_recv, ag_send, ag_recv):
        my = lax.axis_index("i")

        barrier_sem = pltpu.get_barrier_semaphore()
        for k in range(1, N_DEV):
            tgt = lax.rem(my + k, N_DEV)
            pl.semaphore_signal(barrier_sem, inc=1, device_id=(tgt,),
                                device_id_type=pl.DeviceIdType.MESH)

        xf = x_ref[:]
        xb = xf.astype(jnp.bfloat16)
        scores = jnp.dot(xf, router_ref[:], preferred_element_type=jnp.float32)
        s_max = jnp.max(scores, axis=-1, keepdims=True)
        p = jnp.exp(scores - s_max)
        probs = p / jnp.sum(p, axis=-1, keepdims=True)

        wcat = jnp.concatenate(
            [ew_ref[0].astype(jnp.bfloat16), ew_ref[1].astype(jnp.bfloat16)],
            axis=1)
        y = jnp.dot(xb, wcat, preferred_element_type=jnp.float32)

        eids = lax.broadcasted_iota(jnp.int32, (N_TOK, N_EXP), 1)
        partial = jnp.zeros((N_TOK, D_OUT), jnp.float32)
        for j in range(EXP_PER_DEV):
            e = my * EXP_PER_DEV + j
            w = jnp.sum(jnp.where(eids == e, probs, 0.0), axis=1, keepdims=True)
            gate = idx_ref[:] == e
            w = jnp.where(gate, w, 0.0)
            partial = partial + w * y[:, j * D_OUT:(j + 1) * D_OUT]
        ps_ref[:] = partial.astype(jnp.bfloat16)

        pl.semaphore_wait(barrier_sem, N_DEV - 1)

        far_first = list(range(1, N_DEV))
        near_first = list(range(1, N_DEV))

        for k in far_first:
            c = lax.rem(my + k, N_DEV)
            rdma = pltpu.make_async_remote_copy(
                src_ref=ps_ref.at[pl.ds(c * ROWS, ROWS), :],
                dst_ref=rs_buf.at[k - 1],
                send_sem=rs_send.at[k - 1],
                recv_sem=rs_recv.at[k - 1],
                device_id=(c,),
                device_id_type=pl.DeviceIdType.MESH,
            )
            rdma.start()

        out_ref[:] = jnp.dot(xb, sw_ref[:].astype(jnp.bfloat16),
                             preferred_element_type=jnp.float32)

        red = ps_ref[pl.ds(my * ROWS, ROWS), :].astype(jnp.float32)
        for k in near_first:
            wait = pltpu.make_async_remote_copy(
                src_ref=ps_ref.at[pl.ds(0, ROWS), :],
                dst_ref=rs_buf.at[k - 1],
                send_sem=rs_send.at[k - 1],
                recv_sem=rs_recv.at[k - 1],
                device_id=(my,),
                device_id_type=pl.DeviceIdType.MESH,
            )
            wait.wait_recv()
            red = red + rs_buf[k - 1].astype(jnp.float32)
        ag_ref[pl.ds(my * ROWS, ROWS), :] = red.astype(jnp.bfloat16)

        for k in far_first:
            tgt = lax.rem(my + k, N_DEV)
            rdma = pltpu.make_async_remote_copy(
                src_ref=ag_ref.at[pl.ds(my * ROWS, ROWS), :],
                dst_ref=ag_ref.at[pl.ds(my * ROWS, ROWS), :],
                send_sem=ag_send.at[k - 1],
                recv_sem=ag_recv.at[k - 1],
                device_id=(tgt,),
                device_id_type=pl.DeviceIdType.MESH,
            )
            rdma.start()

        for k in near_first:
            wait = pltpu.make_async_remote_copy(
                src_ref=ag_ref.at[pl.ds(0, ROWS), :],
                dst_ref=ag_ref.at[pl.ds(0, ROWS), :],
                send_sem=ag_send.at[k - 1],
                recv_sem=ag_recv.at[k - 1],
                device_id=(my,),
                device_id_type=pl.DeviceIdType.MESH,
            )
            wait.wait_recv()
        for k in range(1, N_DEV):
            wait = pltpu.make_async_remote_copy(
                src_ref=ag_ref.at[pl.ds(0, ROWS), :],
                dst_ref=ag_ref.at[pl.ds(0, ROWS), :],
                send_sem=ag_send.at[k - 1],
                recv_sem=ag_recv.at[k - 1],
                device_id=(my,),
                device_id_type=pl.DeviceIdType.MESH,
            )
            wait.wait_send()
            wait_a = pltpu.make_async_remote_copy(
                src_ref=ps_ref.at[pl.ds(0, ROWS), :],
                dst_ref=rs_buf.at[k - 1],
                send_sem=rs_send.at[k - 1],
                recv_sem=rs_recv.at[k - 1],
                device_id=(my,),
                device_id_type=pl.DeviceIdType.MESH,
            )
            wait_a.wait_send()

        out_ref[:] = out_ref[:] + ag_ref[:].astype(jnp.float32)

    return pl.pallas_call(
        body,
        out_shape=jax.ShapeDtypeStruct((N_TOK, D_OUT), jnp.float32),
        in_specs=[pl.BlockSpec(memory_space=pltpu.VMEM)] * 5,
        out_specs=pl.BlockSpec(memory_space=pltpu.VMEM),
        scratch_shapes=[
            pltpu.VMEM((N_TOK, D_OUT), jnp.bfloat16),
            pltpu.VMEM((N_DEV - 1, ROWS, D_OUT), jnp.bfloat16),
            pltpu.VMEM((N_TOK, D_OUT), jnp.bfloat16),
            pltpu.SemaphoreType.DMA((N_DEV - 1,)),
            pltpu.SemaphoreType.DMA((N_DEV - 1,)),
            pltpu.SemaphoreType.DMA((N_DEV - 1,)),
            pltpu.SemaphoreType.DMA((N_DEV - 1,)),
        ],
        compiler_params=pltpu.CompilerParams(collective_id=0),
    )(x, router_W, route_idx, expert_W, shared_W)
